# baseline (speedup 1.0000x reference)
import jax
import jax.numpy as jnp
from jax.experimental import pallas as pl
from jax.experimental.pallas import tpu as pltpu

_N = 10000
_D = 128
_NPHYS = 5
_BM = 400
_NBUF = 3
_NSUB = 5
_BSUB = _BM // _NSUB


def _signed_log(x, eps=1e-08):
    return jnp.sign(x) * jnp.log(jnp.abs(x) + eps)


def _fused_body(adj_hbm, hfull_ref, hrow_ref, c_ref,
                w1h_ref, w1u_ref, w1p_ref, b1_ref, w2_ref, b2_ref,
                gamma_ref, beta_ref, out_ref, bufs, sems):
    i = pl.program_id(0)
    nsteps = pl.num_programs(0)

    def _copy(blk, slot, sub):
        return pltpu.make_async_copy(
            adj_hbm.at[pl.ds(blk * _BM + sub * _BSUB, _BSUB), :],
            bufs.at[slot, pl.ds(sub * _BSUB, _BSUB), :],
            sems.at[slot])

    def _start(blk, slot):
        for s in range(_NSUB):
            _copy(blk, slot, s).start()

    @pl.when(i == 0)
    def _():
        for k in range(_NBUF - 1):
            _start(k, k)

    @pl.when(i + _NBUF - 1 < nsteps)
    def _():
        blk = i + _NBUF - 1
        _start(blk, jax.lax.rem(blk, _NBUF))

    slot = jax.lax.rem(i, _NBUF)
    for s in range(_NSUB):
        _copy(i, slot, s).wait()

    up = jnp.dot(bufs[slot], hfull_ref[...],
                 preferred_element_type=jnp.float32)

    hrow = hrow_ref[...]
    phys = _signed_log(c_ref[...])

    pre = jnp.dot(hrow, w1h_ref[...], preferred_element_type=jnp.float32)
    pre = pre + jnp.dot(up, w1u_ref[...], preferred_element_type=jnp.float32)
    w1p = w1p_ref[...]
    for j in range(_NPHYS):
        pre = pre + phys[:, j][:, None] * w1p[j, :][None, :]
    pre = pre + b1_ref[...]

    hidden = pre * jax.nn.sigmoid(pre)
    mlp = jnp.dot(hidden, w2_ref[...], preferred_element_type=jnp.float32)
    mlp = mlp + b2_ref[...]

    x = hrow + mlp
    mu = jnp.mean(x, axis=-1, keepdims=True)
    xc = x - mu
    var = jnp.mean(xc * xc, axis=-1, keepdims=True)
    out_ref[...] = gamma_ref[...] * xc * jax.lax.rsqrt(var + 1e-05) \
        + beta_ref[...]


def kernel(h, c1_next_upstream, c2_prev_upstream, c3_self, c4_lateral,
           q_new, adjacency, W1, b1, W2, b2, gamma, beta):
    n, d = h.shape
    c = jnp.stack([c1_next_upstream, c2_prev_upstream, c3_self,
                   c4_lateral, q_new], axis=1)
    w1h = W1[:, :d].T
    w1u = W1[:, d:2 * d].T
    w1p = W1[:, 2 * d:].T
    w2t = W2.T
    b1r = b1.reshape(1, d)
    b2r = b2.reshape(1, d)
    gammar = gamma.reshape(1, d)
    betar = beta.reshape(1, d)

    grid = (n // _BM,)
    out = pl.pallas_call(
        _fused_body,
        grid=grid,
        in_specs=[
            pl.BlockSpec(memory_space=pl.ANY),
            pl.BlockSpec((n, d), lambda i: (0, 0)),
            pl.BlockSpec((_BM, d), lambda i: (i, 0)),
            pl.BlockSpec((_BM, _NPHYS), lambda i: (i, 0)),
            pl.BlockSpec((d, d), lambda i: (0, 0)),
            pl.BlockSpec((d, d), lambda i: (0, 0)),
            pl.BlockSpec((_NPHYS, d), lambda i: (0, 0)),
            pl.BlockSpec((1, d), lambda i: (0, 0)),
            pl.BlockSpec((d, d), lambda i: (0, 0)),
            pl.BlockSpec((1, d), lambda i: (0, 0)),
            pl.BlockSpec((1, d), lambda i: (0, 0)),
            pl.BlockSpec((1, d), lambda i: (0, 0)),
        ],
        out_specs=pl.BlockSpec((_BM, d), lambda i: (i, 0)),
        out_shape=jax.ShapeDtypeStruct((n, d), jnp.float32),
        scratch_shapes=[
            pltpu.VMEM((_NBUF, _BM, _N), jnp.float32),
            pltpu.SemaphoreType.DMA((_NBUF,)),
        ],
    )(adjacency, h, h, c, w1h, w1u, w1p, b1r, w2t, b2r, gammar, betar)
    return out


# speedup vs baseline: 1.0023x; 1.0023x over previous
import jax
import jax.numpy as jnp
from jax.experimental import pallas as pl
from jax.experimental.pallas import tpu as pltpu

_N = 10000
_D = 128
_NPHYS = 5
_BM = 200
_NBUF = 5
_NSUB = 5
_BSUB = _BM // _NSUB


def _signed_log(x, eps=1e-08):
    return jnp.sign(x) * jnp.log(jnp.abs(x) + eps)


def _fused_body(adj_hbm, hfull_ref, hrow_ref, c_ref,
                w1h_ref, w1u_ref, w1p_ref, b1_ref, w2_ref, b2_ref,
                gamma_ref, beta_ref, out_ref, bufs, sems):
    i = pl.program_id(0)
    nsteps = pl.num_programs(0)

    def _copy(blk, slot, sub):
        return pltpu.make_async_copy(
            adj_hbm.at[pl.ds(blk * _BM + sub * _BSUB, _BSUB), :],
            bufs.at[slot, pl.ds(sub * _BSUB, _BSUB), :],
            sems.at[slot])

    def _start(blk, slot):
        for s in range(_NSUB):
            _copy(blk, slot, s).start()

    @pl.when(i == 0)
    def _():
        for k in range(_NBUF - 1):
            _start(k, k)

    @pl.when(i + _NBUF - 1 < nsteps)
    def _():
        blk = i + _NBUF - 1
        _start(blk, jax.lax.rem(blk, _NBUF))

    slot = jax.lax.rem(i, _NBUF)
    for s in range(_NSUB):
        _copy(i, slot, s).wait()

    up = jnp.dot(bufs[slot], hfull_ref[...],
                 preferred_element_type=jnp.float32)

    hrow = hrow_ref[...]
    phys = _signed_log(c_ref[...])

    pre = jnp.dot(hrow, w1h_ref[...], preferred_element_type=jnp.float32)
    pre = pre + jnp.dot(up, w1u_ref[...], preferred_element_type=jnp.float32)
    w1p = w1p_ref[...]
    for j in range(_NPHYS):
        pre = pre + phys[:, j][:, None] * w1p[j, :][None, :]
    pre = pre + b1_ref[...]

    hidden = pre * jax.nn.sigmoid(pre)
    mlp = jnp.dot(hidden, w2_ref[...], preferred_element_type=jnp.float32)
    mlp = mlp + b2_ref[...]

    x = hrow + mlp
    mu = jnp.mean(x, axis=-1, keepdims=True)
    xc = x - mu
    var = jnp.mean(xc * xc, axis=-1, keepdims=True)
    out_ref[...] = gamma_ref[...] * xc * jax.lax.rsqrt(var + 1e-05) \
        + beta_ref[...]


def kernel(h, c1_next_upstream, c2_prev_upstream, c3_self, c4_lateral,
           q_new, adjacency, W1, b1, W2, b2, gamma, beta):
    n, d = h.shape
    c = jnp.stack([c1_next_upstream, c2_prev_upstream, c3_self,
                   c4_lateral, q_new], axis=1)
    w1h = W1[:, :d].T
    w1u = W1[:, d:2 * d].T
    w1p = W1[:, 2 * d:].T
    w2t = W2.T
    b1r = b1.reshape(1, d)
    b2r = b2.reshape(1, d)
    gammar = gamma.reshape(1, d)
    betar = beta.reshape(1, d)

    grid = (n // _BM,)
    out = pl.pallas_call(
        _fused_body,
        grid=grid,
        in_specs=[
            pl.BlockSpec(memory_space=pl.ANY),
            pl.BlockSpec((n, d), lambda i: (0, 0)),
            pl.BlockSpec((_BM, d), lambda i: (i, 0)),
            pl.BlockSpec((_BM, _NPHYS), lambda i: (i, 0)),
            pl.BlockSpec((d, d), lambda i: (0, 0)),
            pl.BlockSpec((d, d), lambda i: (0, 0)),
            pl.BlockSpec((_NPHYS, d), lambda i: (0, 0)),
            pl.BlockSpec((1, d), lambda i: (0, 0)),
            pl.BlockSpec((d, d), lambda i: (0, 0)),
            pl.BlockSpec((1, d), lambda i: (0, 0)),
            pl.BlockSpec((1, d), lambda i: (0, 0)),
            pl.BlockSpec((1, d), lambda i: (0, 0)),
        ],
        out_specs=pl.BlockSpec((_BM, d), lambda i: (i, 0)),
        out_shape=jax.ShapeDtypeStruct((n, d), jnp.float32),
        scratch_shapes=[
            pltpu.VMEM((_NBUF, _BM, _N), jnp.float32),
            pltpu.SemaphoreType.DMA((_NBUF,)),
        ],
    )(adjacency, h, h, c, w1h, w1u, w1p, b1r, w2t, b2r, gammar, betar)
    return out


# speedup vs baseline: 1.0163x; 1.0139x over previous
import jax
import jax.numpy as jnp
from jax.experimental import pallas as pl
from jax.experimental.pallas import tpu as pltpu

_N = 10000
_D = 128
_NPHYS = 5
_BM = 200
_NBUF = 4
_NSUB = 5
_BSUB = _BM // _NSUB


def _signed_log(x, eps=1e-08):
    return jnp.sign(x) * jnp.log(jnp.abs(x) + eps)


def _fused_body(adj_hbm, hfull_ref, hrow_ref, c_ref,
                w1h_ref, w1u_ref, w1p_ref, b1_ref, w2_ref, b2_ref,
                gamma_ref, beta_ref, out_ref, bufs, sems):
    i = pl.program_id(0)
    nsteps = pl.num_programs(0)

    def _copy(blk, slot, sub):
        return pltpu.make_async_copy(
            adj_hbm.at[pl.ds(blk * _BM + sub * _BSUB, _BSUB), :],
            bufs.at[slot, pl.ds(sub * _BSUB, _BSUB), :],
            sems.at[slot])

    def _start(blk, slot):
        for s in range(_NSUB):
            _copy(blk, slot, s).start()

    @pl.when(i == 0)
    def _():
        for k in range(_NBUF - 1):
            _start(k, k)

    @pl.when(i + _NBUF - 1 < nsteps)
    def _():
        blk = i + _NBUF - 1
        _start(blk, jax.lax.rem(blk, _NBUF))

    slot = jax.lax.rem(i, _NBUF)
    for s in range(_NSUB):
        _copy(i, slot, s).wait()

    up = jnp.dot(bufs[slot], hfull_ref[...],
                 preferred_element_type=jnp.float32)

    hrow = hrow_ref[...]
    phys = _signed_log(c_ref[...])

    pre = jnp.dot(hrow, w1h_ref[...], preferred_element_type=jnp.float32)
    pre = pre + jnp.dot(up, w1u_ref[...], preferred_element_type=jnp.float32)
    w1p = w1p_ref[...]
    for j in range(_NPHYS):
        pre = pre + phys[:, j][:, None] * w1p[j, :][None, :]
    pre = pre + b1_ref[...]

    hidden = pre * jax.nn.sigmoid(pre)
    mlp = jnp.dot(hidden, w2_ref[...], preferred_element_type=jnp.float32)
    mlp = mlp + b2_ref[...]

    x = hrow + mlp
    mu = jnp.mean(x, axis=-1, keepdims=True)
    xc = x - mu
    var = jnp.mean(xc * xc, axis=-1, keepdims=True)
    out_ref[...] = gamma_ref[...] * xc * jax.lax.rsqrt(var + 1e-05) \
        + beta_ref[...]


def kernel(h, c1_next_upstream, c2_prev_upstream, c3_self, c4_lateral,
           q_new, adjacency, W1, b1, W2, b2, gamma, beta):
    n, d = h.shape
    c = jnp.stack([c1_next_upstream, c2_prev_upstream, c3_self,
                   c4_lateral, q_new], axis=1)
    w1h = W1[:, :d].T
    w1u = W1[:, d:2 * d].T
    w1p = W1[:, 2 * d:].T
    w2t = W2.T
    b1r = b1.reshape(1, d)
    b2r = b2.reshape(1, d)
    gammar = gamma.reshape(1, d)
    betar = beta.reshape(1, d)

    grid = (n // _BM,)
    out = pl.pallas_call(
        _fused_body,
        grid=grid,
        in_specs=[
            pl.BlockSpec(memory_space=pl.ANY),
            pl.BlockSpec((n, d), lambda i: (0, 0)),
            pl.BlockSpec((_BM, d), lambda i: (i, 0)),
            pl.BlockSpec((_BM, _NPHYS), lambda i: (i, 0)),
            pl.BlockSpec((d, d), lambda i: (0, 0)),
            pl.BlockSpec((d, d), lambda i: (0, 0)),
            pl.BlockSpec((_NPHYS, d), lambda i: (0, 0)),
            pl.BlockSpec((1, d), lambda i: (0, 0)),
            pl.BlockSpec((d, d), lambda i: (0, 0)),
            pl.BlockSpec((1, d), lambda i: (0, 0)),
            pl.BlockSpec((1, d), lambda i: (0, 0)),
            pl.BlockSpec((1, d), lambda i: (0, 0)),
        ],
        out_specs=pl.BlockSpec((_BM, d), lambda i: (i, 0)),
        out_shape=jax.ShapeDtypeStruct((n, d), jnp.float32),
        scratch_shapes=[
            pltpu.VMEM((_NBUF, _BM, _N), jnp.float32),
            pltpu.SemaphoreType.DMA((_NBUF,)),
        ],
    )(adjacency, h, h, c, w1h, w1u, w1p, b1r, w2t, b2r, gammar, betar)
    return out
